# speedup vs baseline: 10.7667x; 10.7667x over previous
import functools
import math

import jax
import jax.numpy as jnp
from jax.experimental import pallas as pl

B, P, D, T, TOP_K = 4096, 64, 2048, 8, 2
KBLK = 2048
BBLK = 512


def _pv_matmul_kernel(pv_ref, w_ref, out_ref):
    k = pl.program_id(0)

    @pl.when(k == 0)
    def _init():
        out_ref[...] = jnp.zeros_like(out_ref)

    out_ref[...] += jnp.dot(pv_ref[...], w_ref[...],
                            preferred_element_type=jnp.float32)


def _fused_router_kernel(q_ref, k_ref, pv_ref, t_ref, b_ref, g_ref, be_ref,
                         out_ref, idx_ref):
    q = q_ref[...]
    kk = k_ref[...]

    qn = q / jnp.maximum(jnp.sqrt(jnp.sum(q * q, axis=1, keepdims=True)), 1e-12)
    kn = kk / jnp.maximum(jnp.sqrt(jnp.sum(kk * kk, axis=1, keepdims=True)), 1e-12)

    temp = jnp.clip(t_ref[0, 0], 0.1, 2.0)
    sim = jnp.dot(qn, kn.T, preferred_element_type=jnp.float32) / temp

    col = jax.lax.broadcasted_iota(jnp.int32, sim.shape, 1)
    m0 = jnp.max(sim, axis=1, keepdims=True)
    i0 = jnp.min(jnp.where(sim >= m0, col, P), axis=1, keepdims=True)
    masked = jnp.where(col == i0, -jnp.inf, sim)
    m1 = jnp.max(masked, axis=1, keepdims=True)
    i1 = jnp.min(jnp.where(masked >= m1, col, P), axis=1, keepdims=True)

    e = jnp.exp(m1 - m0)
    w0 = 1.0 / (1.0 + e)
    w1 = e / (1.0 + e)

    a = jnp.where(col == i0, w0, 0.0) + jnp.where(col == i1, w1, 0.0)
    h = jnp.dot(a, pv_ref[...], preferred_element_type=jnp.float32) + b_ref[...]

    mu = jnp.mean(h, axis=1, keepdims=True)
    var = jnp.mean((h - mu) * (h - mu), axis=1, keepdims=True)
    hn = (h - mu) / jnp.sqrt(var + 1e-5) * g_ref[...] + be_ref[...]

    out_ref[...] = 0.5 * hn * (1.0 + jax.lax.erf(hn * (1.0 / math.sqrt(2.0))))
    idx_ref[...] = jnp.concatenate(
        [i0.astype(jnp.int32), i1.astype(jnp.int32)], axis=1)


@jax.jit
def kernel(query_feature, prompt_keys, prompt_values, temperature, W, b,
           gamma, beta):
    pv_flat = prompt_values.reshape(P, T * D)

    pv_table = pl.pallas_call(
        _pv_matmul_kernel,
        grid=(T * D // KBLK,),
        in_specs=[
            pl.BlockSpec((P, KBLK), lambda k: (0, k)),
            pl.BlockSpec((KBLK, D), lambda k: (k, 0)),
        ],
        out_specs=pl.BlockSpec((P, D), lambda k: (0, 0)),
        out_shape=jax.ShapeDtypeStruct((P, D), jnp.float32),
    )(pv_flat, W)

    t2 = jnp.asarray(temperature, jnp.float32).reshape(1, 1)
    out, idx = pl.pallas_call(
        _fused_router_kernel,
        grid=(B // BBLK,),
        in_specs=[
            pl.BlockSpec((BBLK, D), lambda i: (i, 0)),
            pl.BlockSpec((P, D), lambda i: (0, 0)),
            pl.BlockSpec((P, D), lambda i: (0, 0)),
            pl.BlockSpec((1, 1), lambda i: (0, 0)),
            pl.BlockSpec((1, D), lambda i: (0, 0)),
            pl.BlockSpec((1, D), lambda i: (0, 0)),
            pl.BlockSpec((1, D), lambda i: (0, 0)),
        ],
        out_specs=[
            pl.BlockSpec((BBLK, D), lambda i: (i, 0)),
            pl.BlockSpec((BBLK, TOP_K), lambda i: (i, 0)),
        ],
        out_shape=[
            jax.ShapeDtypeStruct((B, D), jnp.float32),
            jax.ShapeDtypeStruct((B, TOP_K), jnp.int32),
        ],
    )(query_feature, prompt_keys, pv_table, t2, b.reshape(1, D),
      gamma.reshape(1, D), beta.reshape(1, D))

    return (out, idx)
